# baseline (speedup 1.0000x reference)
import functools

import jax
import jax.numpy as jnp
from jax import lax
from jax.experimental import pallas as pl
from jax.experimental.pallas import tpu as pltpu
from jax.experimental.pallas import tpu_sc as plsc

_D = 64
_L = 16
_NC = 2
_NS = 16
_NW = _NC * _NS


def _transpose_block(gbuf, tbuf):
    lanes = lax.iota(jnp.int32, _L)
    for r in range(128):
        for j in range(_D // _L):
            v = gbuf[r, pl.ds(j * _L, _L)]
            plsc.store_scatter(tbuf, [j * _L + lanes,
                                      jnp.full((_L,), r, jnp.int32)], v)


@functools.lru_cache(maxsize=None)
def _build(B: int, T: int, D: int):
    bw = B // _NW
    mesh = plsc.VectorSubcoreMesh(core_axis_name="c", subcore_axis_name="s")

    @functools.partial(
        pl.kernel,
        mesh=mesh,
        out_type=jax.ShapeDtypeStruct((T, D, B), jnp.float32),
        scratch_types=[
            pltpu.VMEM((bw,), jnp.int32),
            pltpu.VMEM((bw,), jnp.int32),
            pltpu.VMEM((bw, D), jnp.float32),
            pltpu.VMEM((bw, D), jnp.float32),
            pltpu.VMEM((D, bw + 1), jnp.float32),
            pltpu.VMEM((D, bw + 1), jnp.float32),
            pltpu.SemaphoreType.DMA((2,)),
            pltpu.SemaphoreType.DMA((2,)),
        ],
        compiler_params=pltpu.CompilerParams(use_tc_tiling_on_sc=False,
                                             needs_layout_passes=False),
    )
    def gather_kernel(table_hbm, idx_hbm, x_hbm, idx_v0, idx_v1,
                      gbuf0, gbuf1, tbuf0, tbuf1, gsem, wsem):
        idx_v = [idx_v0, idx_v1]
        gbuf = [gbuf0, gbuf1]
        tbuf = [tbuf0, tbuf1]
        wid = lax.axis_index("s") * _NC + lax.axis_index("c")
        b0 = wid * bw

        def load_idx(t, b):
            pltpu.sync_copy(idx_hbm.at[pl.ds(t * B + b0, bw)], idx_v[b])

        def start_gather(b):
            pltpu.async_copy(table_hbm.at[idx_v[b]], gbuf[b], gsem.at[b])

        def wait_gather(b):
            pltpu.make_async_copy(table_hbm.at[idx_v[b]], gbuf[b],
                                  gsem.at[b]).wait()

        def start_write(t, b):
            pltpu.async_copy(tbuf[b].at[:, pl.ds(0, bw)],
                             x_hbm.at[t, :, pl.ds(b0, bw)], wsem.at[b])

        def wait_write(t, b):
            pltpu.make_async_copy(tbuf[b].at[:, pl.ds(0, bw)],
                                  x_hbm.at[t, :, pl.ds(b0, bw)],
                                  wsem.at[b]).wait()

        for b in range(2):
            load_idx(b, b)
            start_gather(b)

        for b in range(2):
            wait_gather(b)
            _transpose_block(gbuf[b], tbuf[b])
            start_write(b, b)
            load_idx(b + 2, b)
            start_gather(b)

        @pl.loop(2, T - 2, step=2)
        def _pair(g):
            for b in range(2):
                t = g + b
                wait_gather(b)
                wait_write(t - 2, b)
                _transpose_block(gbuf[b], tbuf[b])
                start_write(t, b)
                load_idx(t + 2, b)
                start_gather(b)

        for b in range(2):
            t = T - 2 + b
            wait_gather(b)
            wait_write(t - 2, b)
            _transpose_block(gbuf[b], tbuf[b])
            start_write(t, b)
        for b in range(2):
            wait_write(T - 2 + b, b)

    return gather_kernel


def kernel(seq, table):
    s0, s1 = seq.shape
    tidx = seq.T.reshape(s0 * s1).astype(jnp.int32)
    x = _build(s0, s1, _D)(table, tidx)
    return x.transpose(2, 0, 1)


# speedup vs baseline: 1.4766x; 1.4766x over previous
import functools

import jax
import jax.numpy as jnp
from jax import lax
from jax.experimental import pallas as pl
from jax.experimental.pallas import tpu as pltpu
from jax.experimental.pallas import tpu_sc as plsc

_D = 64
_DP = 128
_NC = 2
_NS = 16
_NW = _NC * _NS


@functools.lru_cache(maxsize=None)
def _build(B: int, C: int):
    b_per_w = B // _NW
    n = b_per_w // C
    assert n % 2 == 0 and n >= 4
    mesh = plsc.VectorSubcoreMesh(core_axis_name="c", subcore_axis_name="s")

    @functools.partial(
        pl.kernel,
        mesh=mesh,
        out_type=jax.ShapeDtypeStruct((B, _DP), jnp.float32),
        scratch_types=[
            pltpu.VMEM((C,), jnp.int32),
            pltpu.VMEM((C,), jnp.int32),
            pltpu.VMEM((C, _DP), jnp.float32),
            pltpu.VMEM((C, _DP), jnp.float32),
            pltpu.SemaphoreType.DMA((2,)),
            pltpu.SemaphoreType.DMA((2,)),
        ],
    )
    def gather_kernel(table_hbm, idx_hbm, out_hbm, idx_v0, idx_v1,
                      rows_v0, rows_v1, gsem, wsem):
        idx_v = [idx_v0, idx_v1]
        rows_v = [rows_v0, rows_v1]
        wid = lax.axis_index("s") * _NC + lax.axis_index("c")
        base = wid * b_per_w

        for b in range(2):
            pltpu.sync_copy(idx_hbm.at[pl.ds(base + b * C, C)], idx_v[b])
            pltpu.async_copy(table_hbm.at[idx_v[b]], rows_v[b],
                             gsem.at[b])

        @pl.loop(0, n - 2, step=2)
        def _pair(g):
            for b in range(2):
                i = g + b
                off = base + i * C
                pltpu.make_async_copy(table_hbm.at[idx_v[b]],
                                      rows_v[b], gsem.at[b]).wait()
                pltpu.async_copy(rows_v[b],
                                 out_hbm.at[pl.ds(off, C)], wsem.at[b])
                pltpu.sync_copy(idx_hbm.at[pl.ds(off + 2 * C, C)],
                                idx_v[b])
                pltpu.make_async_copy(rows_v[b],
                                      out_hbm.at[pl.ds(off, C)],
                                      wsem.at[b]).wait()
                pltpu.async_copy(table_hbm.at[idx_v[b]], rows_v[b],
                                 gsem.at[b])

        for b in range(2):
            i = n - 2 + b
            off = base + i * C
            pltpu.make_async_copy(table_hbm.at[idx_v[b]], rows_v[b],
                                  gsem.at[b]).wait()
            pltpu.sync_copy(rows_v[b], out_hbm.at[pl.ds(off, C)])

    return gather_kernel


def kernel(seq, table):
    s0, s1 = seq.shape
    b = s0 * s1
    flat = seq.reshape(b).astype(jnp.int32)
    tp = jnp.pad(table, ((0, 0), (0, _DP - _D)))
    out = _build(b, 400)(tp, flat)
    return out[:, :_D].reshape(s0, s1, _D)
